# baseline (speedup 1.0000x reference)
import numpy as np
import jax
import jax.numpy as jnp
from jax import lax
from jax.experimental import pallas as pl
from jax.experimental.pallas import tpu as pltpu

B, N, VD, MD, ID, K, H = 4, 4096, 1024, 256, 1024, 1024, 16
DH = ID // H
G = N // K

_IDX = np.linspace(0, N - 1, K).astype(np.int64)
_OFF = _IDX - G * np.arange(K)
if not ((_OFF >= 0).all() and (_OFF < G).all()
        and np.all(np.isin(np.diff(_OFF), [0, 1]))):
    raise ValueError("anchor index structure unexpected")
_BOUNDS = [int(x) for x in (np.where(np.diff(_OFF) != 0)[0] + 1)]

_CH = 256
_HG = 8
_RB = 2048


def _seg(i):
    s = i * 0 if hasattr(i, "shape") else 0
    for bnd in _BOUNDS:
        s = s + (i >= bnd)
    return s


def _select_anchors(blk, j):
    d = blk.shape[-1]
    x4 = blk.reshape(_CH, G, d)
    i = lax.broadcasted_iota(jnp.int32, (_CH, 1), 0) + j * _CH
    seg = _seg(i)
    out = x4[:, 0, :] * (seg == 0).astype(blk.dtype)
    for o in range(1, G):
        out = out + x4[:, o, :] * (seg == o).astype(blk.dtype)
    return out


def _cast_kernel(wq, wk, wv, wo, wq16, wk16, wv16, wo16):
    bf16 = jnp.bfloat16
    wq16[...] = wq[...].astype(bf16)
    wk16[...] = wk[...].astype(bf16)
    wv16[...] = wv[...].astype(bf16)
    wo16[...] = wo[...].astype(bf16)


def _gp_kernel(va, ma, vb, mb, wq, wk, wv,
               qa, ka, ua, qb, kb, ub):
    j = pl.program_id(1)
    f32 = jnp.float32
    bf16 = jnp.bfloat16
    scale = 1.0 / np.sqrt(DH)
    wqv, wqm = wq[0:VD, :], wq[VD:, :]
    wkv, wkm = wk[0:VD, :], wk[VD:, :]
    for v_in, m_in, q_o, k_o, v_o in ((va, ma, qa, ka, ua),
                                      (vb, mb, qb, kb, ub)):
        xv = _select_anchors(v_in[0].astype(bf16), j)
        xm = _select_anchors(m_in[0].astype(bf16), j)
        q = (jnp.dot(xv, wqv, preferred_element_type=f32)
             + jnp.dot(xm, wqm, preferred_element_type=f32)) * scale
        k = (jnp.dot(xv, wkv, preferred_element_type=f32)
             + jnp.dot(xm, wkm, preferred_element_type=f32))
        v = jnp.dot(xv, wv[...], preferred_element_type=f32)
        q_o[0] = q.astype(bf16)
        k_o[0] = k.astype(bf16)
        v_o[0] = v.astype(bf16)


def _attn_kernel(qa, ka, ua, qb, kb, ub, wo, out):
    hg = pl.program_id(1)
    f32 = jnp.float32
    bf16 = jnp.bfloat16
    q = jnp.concatenate([qa[0], qb[0]], axis=0)
    k = jnp.concatenate([ka[0], kb[0]], axis=0)
    v = jnp.concatenate([ua[0], ub[0]], axis=0)
    ohs = []
    for hh in range(_HG):
        sl = slice(hh * DH, (hh + 1) * DH)
        qh, kh, vh = q[:, sl], k[:, sl], v[:, sl]
        s = lax.dot_general(qh, kh, (((1,), (1,)), ((), ())),
                            preferred_element_type=f32)
        e = jnp.exp(s).astype(bf16)
        v_aug = jnp.concatenate([vh, jnp.ones_like(vh)], axis=1)
        oh_aug = jnp.dot(e, v_aug, preferred_element_type=f32)
        ohs.append((oh_aug[:, :DH] / oh_aug[:, DH:]).astype(bf16))
    acc = jnp.dot(jnp.concatenate(ohs, axis=1), wo[...],
                  preferred_element_type=f32)

    @pl.when(hg == 0)
    def _():
        out[0] = acc

    @pl.when(hg != 0)
    def _():
        out[0] = out[0] + acc


def _combine_kernel(v_in, u, out):
    j = pl.program_id(1)
    r = lax.broadcasted_iota(jnp.int32, (_RB, 1), 0) + j * _RB
    i = r // G
    sel = (r % G) == _seg(i)
    nsl = _RB // G
    u_exp = jnp.broadcast_to(u[0][:, None, :],
                             (nsl, G, VD)).reshape(_RB, VD)
    out[0] = v_in[0] + jnp.where(sel, u_exp, 0.0)


def _combine(values, u, half):
    jblocks = N // _RB
    nsl = _RB // G
    return pl.pallas_call(
        _combine_kernel,
        grid=(B, jblocks),
        in_specs=[
            pl.BlockSpec((1, _RB, VD), lambda b, j: (b, j, 0)),
            pl.BlockSpec((1, nsl, VD),
                         lambda b, j, h=half: (b, h * (K // nsl) + j, 0)),
        ],
        out_specs=pl.BlockSpec((1, _RB, VD), lambda b, j: (b, j, 0)),
        out_shape=jax.ShapeDtypeStruct((B, N, VD), jnp.float32),
        compiler_params=pltpu.CompilerParams(
            dimension_semantics=("arbitrary", "arbitrary"),
        ),
    )(values, u)


def kernel(values_a, metadata_a, values_b, metadata_b, Wq, Wk, Wv, Wo):
    bf16 = jnp.bfloat16

    full = lambda a: pl.BlockSpec(a.shape, lambda: (0,) * a.ndim)
    Wq16, Wk16, Wv16, Wo16 = pl.pallas_call(
        _cast_kernel,
        in_specs=[full(Wq), full(Wk), full(Wv), full(Wo)],
        out_specs=[full(Wq), full(Wk), full(Wv), full(Wo)],
        out_shape=[jax.ShapeDtypeStruct(a.shape, bf16)
                   for a in (Wq, Wk, Wv, Wo)],
    )(Wq, Wk, Wv, Wo)

    rows = G * _CH
    vspec = pl.BlockSpec((1, rows, VD), lambda b, j: (b, j, 0))
    mspec = pl.BlockSpec((1, rows, MD), lambda b, j: (b, j, 0))
    wspec = lambda r, c: pl.BlockSpec((r, c), lambda b, j: (0, 0))
    kid = jax.ShapeDtypeStruct((B, K, ID), bf16)
    qkv_out = pl.BlockSpec((1, _CH, ID), lambda b, j: (b, j, 0))
    qa, ka, ua, qb, kb, ub = pl.pallas_call(
        _gp_kernel,
        grid=(B, K // _CH),
        in_specs=[
            vspec, mspec, vspec, mspec,
            wspec(VD + MD, ID), wspec(VD + MD, ID), wspec(VD, ID),
        ],
        out_specs=[qkv_out] * 6,
        out_shape=[kid] * 6,
        compiler_params=pltpu.CompilerParams(
            dimension_semantics=("arbitrary", "arbitrary"),
            vmem_limit_bytes=63 * 1024 * 1024,
        ),
    )(values_a, metadata_a, values_b, metadata_b, Wq16, Wk16, Wv16)

    hw = _HG * DH
    head_in = pl.BlockSpec((1, K, hw), lambda b, hg: (b, 0, hg))
    u = pl.pallas_call(
        _attn_kernel,
        grid=(B, H // _HG),
        in_specs=[head_in] * 6 + [
            pl.BlockSpec((hw, VD), lambda b, hg: (hg, 0)),
        ],
        out_specs=pl.BlockSpec((1, 2 * K, VD), lambda b, hg: (b, 0, 0)),
        out_shape=jax.ShapeDtypeStruct((B, 2 * K, VD), jnp.float32),
        compiler_params=pltpu.CompilerParams(
            dimension_semantics=("arbitrary", "arbitrary"),
            vmem_limit_bytes=63 * 1024 * 1024,
        ),
    )(qa, ka, ua, qb, kb, ub, Wo16)

    out_a = _combine(values_a, u, 0)
    out_b = _combine(values_b, u, 1)
    return out_a, out_b


# speedup vs baseline: 1.2038x; 1.2038x over previous
import functools

import numpy as np
import jax
import jax.numpy as jnp
from jax import lax
from jax.experimental import pallas as pl
from jax.experimental.pallas import tpu as pltpu
from jax.experimental.pallas import tpu_sc as plsc

B, N, VD, MD, ID, K, H = 4, 4096, 1024, 256, 1024, 1024, 16
DH = ID // H
G = N // K

_IDX = np.linspace(0, N - 1, K).astype(np.int64)
_OFF = _IDX - G * np.arange(K)
if not ((_OFF >= 0).all() and (_OFF < G).all()
        and np.all(np.isin(np.diff(_OFF), [0, 1]))):
    raise ValueError("anchor index structure unexpected")
_BOUNDS = [int(x) for x in (np.where(np.diff(_OFF) != 0)[0] + 1)]

_CH = 256
_HG = 4
_RB = 2048


def _seg(i):
    s = i * 0 if hasattr(i, "shape") else 0
    for bnd in _BOUNDS:
        s = s + (i >= bnd)
    return s


def _select_anchors(blk, j):
    d = blk.shape[-1]
    x4 = blk.reshape(_CH, G, d)
    i = lax.broadcasted_iota(jnp.int32, (_CH, 1), 0) + j * _CH
    seg = _seg(i)
    out = x4[:, 0, :] * (seg == 0).astype(blk.dtype)
    for o in range(1, G):
        out = out + x4[:, o, :] * (seg == o).astype(blk.dtype)
    return out


_IDX_FLAT = (np.arange(B)[:, None] * N + _IDX[None, :]).reshape(-1)
_IDX_FLAT = _IDX_FLAT.astype(np.int32)


def _sc_gather(values_a, metadata_a, values_b, metadata_b, idx_flat):
    info = plsc.get_sparse_core_info()
    nw = info.num_cores * info.num_subcores
    rows_total = B * K
    bpw = rows_total // nw
    cv = 64
    mesh = plsc.VectorSubcoreMesh(core_axis_name="c", subcore_axis_name="s")
    f32 = jnp.float32

    @functools.partial(
        pl.kernel, mesh=mesh,
        out_type=[jax.ShapeDtypeStruct((rows_total, VD), f32),
                  jax.ShapeDtypeStruct((rows_total, MD), f32),
                  jax.ShapeDtypeStruct((rows_total, VD), f32),
                  jax.ShapeDtypeStruct((rows_total, MD), f32)],
        scratch_types=[
            pltpu.VMEM((bpw,), jnp.int32),
            pltpu.VMEM((cv,), jnp.int32),
            pltpu.VMEM((cv, VD), f32),
            pltpu.VMEM((bpw, MD), f32),
            pltpu.SemaphoreType.DMA,
        ],
    )
    def gather(va, ma, vb, mb, idx_hbm, xva, xma, xvb, xmb,
               idx_v, idx_c, rows_v, rows_m, sem):
        wid = lax.axis_index("s") * info.num_cores + lax.axis_index("c")
        base = wid * bpw
        pltpu.sync_copy(idx_hbm.at[pl.ds(base, bpw)], idx_v)
        for vt, mt, vo, mo in ((va, ma, xva, xma), (vb, mb, xvb, xmb)):
            for c in range(bpw // cv):
                pltpu.sync_copy(idx_hbm.at[pl.ds(base + c * cv, cv)], idx_c)
                pltpu.async_copy(vt.at[idx_c], rows_v, sem).wait()
                pltpu.sync_copy(rows_v, vo.at[pl.ds(base + c * cv, cv)])
            pltpu.async_copy(mt.at[idx_v], rows_m, sem).wait()
            pltpu.sync_copy(rows_m, mo.at[pl.ds(base, bpw)])

    return gather(values_a.reshape(B * N, VD),
                  metadata_a.reshape(B * N, MD),
                  values_b.reshape(B * N, VD),
                  metadata_b.reshape(B * N, MD),
                  idx_flat)


def _cast_kernel(wq, wk, wv, wo, wq16, wk16, wv16, wo16):
    bf16 = jnp.bfloat16
    wq16[...] = wq[...].astype(bf16)
    wk16[...] = wk[...].astype(bf16)
    wv16[...] = wv[...].astype(bf16)
    wo16[...] = wo[...].astype(bf16)


def _gp_kernel(va, ma, vb, mb, wq, wk, wv,
               qa, ka, ua, qb, kb, ub):
    f32 = jnp.float32
    bf16 = jnp.bfloat16
    scale = 1.0 / np.sqrt(DH)
    wqv, wqm = wq[0:VD, :], wq[VD:, :]
    wkv, wkm = wk[0:VD, :], wk[VD:, :]
    for v_in, m_in, q_o, k_o, v_o in ((va, ma, qa, ka, ua),
                                      (vb, mb, qb, kb, ub)):
        xv = v_in[0].astype(bf16)
        xm = m_in[0].astype(bf16)
        q = (jnp.dot(xv, wqv, preferred_element_type=f32)
             + jnp.dot(xm, wqm, preferred_element_type=f32)) * scale
        k = (jnp.dot(xv, wkv, preferred_element_type=f32)
             + jnp.dot(xm, wkm, preferred_element_type=f32))
        v = jnp.dot(xv, wv[...], preferred_element_type=f32)
        q_o[0] = q.astype(bf16)
        k_o[0] = k.astype(bf16)
        v_o[0] = v.astype(bf16)


def _attn_kernel(qa, ka, ua, qb, kb, ub, wo, out):
    hg = pl.program_id(1)
    f32 = jnp.float32
    bf16 = jnp.bfloat16
    q = jnp.concatenate([qa[0], qb[0]], axis=0)
    k = jnp.concatenate([ka[0], kb[0]], axis=0)
    v = jnp.concatenate([ua[0], ub[0]], axis=0)
    ohs = []
    for hh in range(_HG):
        sl = slice(hh * DH, (hh + 1) * DH)
        qh, kh, vh = q[:, sl], k[:, sl], v[:, sl]
        s = lax.dot_general(qh, kh, (((1,), (1,)), ((), ())),
                            preferred_element_type=f32)
        e = jnp.exp(s).astype(bf16)
        v_aug = jnp.concatenate([vh, jnp.ones_like(vh)], axis=1)
        oh_aug = jnp.dot(e, v_aug, preferred_element_type=f32)
        ohs.append((oh_aug[:, :DH] / oh_aug[:, DH:]).astype(bf16))
    acc = jnp.dot(jnp.concatenate(ohs, axis=1), wo[...],
                  preferred_element_type=f32)

    @pl.when(hg == 0)
    def _():
        out[0] = acc

    @pl.when(hg != 0)
    def _():
        out[0] = out[0] + acc


def _combine_kernel(v_in, u, out):
    j = pl.program_id(1)
    r = lax.broadcasted_iota(jnp.int32, (_RB, 1), 0) + j * _RB
    i = r // G
    sel = (r % G) == _seg(i)
    nsl = _RB // G
    u_exp = jnp.broadcast_to(u[0][:, None, :],
                             (nsl, G, VD)).reshape(_RB, VD)
    out[0] = v_in[0] + jnp.where(sel, u_exp, 0.0)


def _combine(values, u, half):
    jblocks = N // _RB
    nsl = _RB // G
    return pl.pallas_call(
        _combine_kernel,
        grid=(B, jblocks),
        in_specs=[
            pl.BlockSpec((1, _RB, VD), lambda b, j: (b, j, 0)),
            pl.BlockSpec((1, nsl, VD),
                         lambda b, j, h=half: (b, h * (K // nsl) + j, 0)),
        ],
        out_specs=pl.BlockSpec((1, _RB, VD), lambda b, j: (b, j, 0)),
        out_shape=jax.ShapeDtypeStruct((B, N, VD), jnp.float32),
        compiler_params=pltpu.CompilerParams(
            dimension_semantics=("arbitrary", "arbitrary"),
        ),
    )(values, u)


def kernel(values_a, metadata_a, values_b, metadata_b, Wq, Wk, Wv, Wo):
    bf16 = jnp.bfloat16

    full = lambda a: pl.BlockSpec(a.shape, lambda: (0,) * a.ndim)
    Wq16, Wk16, Wv16, Wo16 = pl.pallas_call(
        _cast_kernel,
        in_specs=[full(Wq), full(Wk), full(Wv), full(Wo)],
        out_specs=[full(Wq), full(Wk), full(Wv), full(Wo)],
        out_shape=[jax.ShapeDtypeStruct(a.shape, bf16)
                   for a in (Wq, Wk, Wv, Wo)],
    )(Wq, Wk, Wv, Wo)

    xva, xma, xvb, xmb = _sc_gather(values_a, metadata_a,
                                    values_b, metadata_b,
                                    jnp.asarray(_IDX_FLAT))

    vspec = pl.BlockSpec((1, _CH, VD), lambda b, j: (b, j, 0))
    mspec = pl.BlockSpec((1, _CH, MD), lambda b, j: (b, j, 0))
    wspec = lambda r, c: pl.BlockSpec((r, c), lambda b, j: (0, 0))
    kid = jax.ShapeDtypeStruct((B, K, ID), bf16)
    qkv_out = pl.BlockSpec((1, _CH, ID), lambda b, j: (b, j, 0))
    qa, ka, ua, qb, kb, ub = pl.pallas_call(
        _gp_kernel,
        grid=(B, K // _CH),
        in_specs=[
            vspec, mspec, vspec, mspec,
            wspec(VD + MD, ID), wspec(VD + MD, ID), wspec(VD, ID),
        ],
        out_specs=[qkv_out] * 6,
        out_shape=[kid] * 6,
        compiler_params=pltpu.CompilerParams(
            dimension_semantics=("arbitrary", "arbitrary"),
            vmem_limit_bytes=63 * 1024 * 1024,
        ),
    )(xva.reshape(B, K, VD), xma.reshape(B, K, MD),
      xvb.reshape(B, K, VD), xmb.reshape(B, K, MD), Wq16, Wk16, Wv16)

    hw = _HG * DH
    head_in = pl.BlockSpec((1, K, hw), lambda b, hg: (b, 0, hg))
    u = pl.pallas_call(
        _attn_kernel,
        grid=(B, H // _HG),
        in_specs=[head_in] * 6 + [
            pl.BlockSpec((hw, VD), lambda b, hg: (hg, 0)),
        ],
        out_specs=pl.BlockSpec((1, 2 * K, VD), lambda b, hg: (b, 0, 0)),
        out_shape=jax.ShapeDtypeStruct((B, 2 * K, VD), jnp.float32),
        compiler_params=pltpu.CompilerParams(
            dimension_semantics=("arbitrary", "arbitrary"),
            vmem_limit_bytes=63 * 1024 * 1024,
        ),
    )(qa, ka, ua, qb, kb, ub, Wo16)

    out_a = _combine(values_a, u, 0)
    out_b = _combine(values_b, u, 1)
    return out_a, out_b


# speedup vs baseline: 1.2113x; 1.0063x over previous
import functools

import numpy as np
import jax
import jax.numpy as jnp
from jax import lax
from jax.experimental import pallas as pl
from jax.experimental.pallas import tpu as pltpu
from jax.experimental.pallas import tpu_sc as plsc

B, N, VD, MD, ID, K, H = 4, 4096, 1024, 256, 1024, 1024, 16
DH = ID // H
G = N // K

_IDX = np.linspace(0, N - 1, K).astype(np.int64)
_OFF = _IDX - G * np.arange(K)
if not ((_OFF >= 0).all() and (_OFF < G).all()
        and np.all(np.isin(np.diff(_OFF), [0, 1]))):
    raise ValueError("anchor index structure unexpected")
_BOUNDS = [int(x) for x in (np.where(np.diff(_OFF) != 0)[0] + 1)]

_CH = 256
_HG = 4
_RB = 2048


def _seg(i):
    s = i * 0 if hasattr(i, "shape") else 0
    for bnd in _BOUNDS:
        s = s + (i >= bnd)
    return s


def _select_anchors(blk, j):
    d = blk.shape[-1]
    x4 = blk.reshape(_CH, G, d)
    i = lax.broadcasted_iota(jnp.int32, (_CH, 1), 0) + j * _CH
    seg = _seg(i)
    out = x4[:, 0, :] * (seg == 0).astype(blk.dtype)
    for o in range(1, G):
        out = out + x4[:, o, :] * (seg == o).astype(blk.dtype)
    return out


_IDX_FLAT = (np.arange(B)[:, None] * N + _IDX[None, :]).reshape(-1)
_IDX_FLAT = _IDX_FLAT.astype(np.int32)


def _sc_gather(values, metadata, idx_flat):
    info = plsc.get_sparse_core_info()
    nw = info.num_cores * info.num_subcores
    rows_total = B * K
    bpw = rows_total // nw
    cv = 64
    mesh = plsc.VectorSubcoreMesh(core_axis_name="c", subcore_axis_name="s")
    f32 = jnp.float32

    @functools.partial(
        pl.kernel, mesh=mesh,
        out_type=[jax.ShapeDtypeStruct((rows_total, VD), f32),
                  jax.ShapeDtypeStruct((rows_total, MD), f32)],
        scratch_types=[
            pltpu.VMEM((bpw,), jnp.int32),
            pltpu.VMEM((cv,), jnp.int32),
            pltpu.VMEM((cv, VD), f32),
            pltpu.VMEM((bpw, MD), f32),
            pltpu.SemaphoreType.DMA,
        ],
    )
    def gather(vt, mt, idx_hbm, vo, mo, idx_v, idx_c, rows_v, rows_m, sem):
        wid = lax.axis_index("s") * info.num_cores + lax.axis_index("c")
        base = wid * bpw
        pltpu.sync_copy(idx_hbm.at[pl.ds(base, bpw)], idx_v)
        for c in range(bpw // cv):
            pltpu.sync_copy(idx_hbm.at[pl.ds(base + c * cv, cv)], idx_c)
            pltpu.async_copy(vt.at[idx_c], rows_v, sem).wait()
            pltpu.sync_copy(rows_v, vo.at[pl.ds(base + c * cv, cv)])
        pltpu.async_copy(mt.at[idx_v], rows_m, sem).wait()
        pltpu.sync_copy(rows_m, mo.at[pl.ds(base, bpw)])

    return gather(values.reshape(B * N, VD),
                  metadata.reshape(B * N, MD),
                  idx_flat)


def _cast_kernel(wq, wk, wv, wo, wq16, wk16, wv16, wo16):
    bf16 = jnp.bfloat16
    wq16[...] = wq[...].astype(bf16)
    wk16[...] = wk[...].astype(bf16)
    wv16[...] = wv[...].astype(bf16)
    wo16[...] = wo[...].astype(bf16)


def _gp_kernel(v_in, m_in, wq, wk, wv, q_o, k_o, v_o):
    f32 = jnp.float32
    bf16 = jnp.bfloat16
    scale = 1.0 / np.sqrt(DH)
    wqv, wqm = wq[0:VD, :], wq[VD:, :]
    wkv, wkm = wk[0:VD, :], wk[VD:, :]
    xv = v_in[0].astype(bf16)
    xm = m_in[0].astype(bf16)
    q = (jnp.dot(xv, wqv, preferred_element_type=f32)
         + jnp.dot(xm, wqm, preferred_element_type=f32)) * scale
    k = (jnp.dot(xv, wkv, preferred_element_type=f32)
         + jnp.dot(xm, wkm, preferred_element_type=f32))
    v = jnp.dot(xv, wv[...], preferred_element_type=f32)
    q_o[0] = q.astype(bf16)
    k_o[0] = k.astype(bf16)
    v_o[0] = v.astype(bf16)


def _attn_kernel(qa, ka, ua, qb, kb, ub, wo, out):
    hg = pl.program_id(1)
    f32 = jnp.float32
    bf16 = jnp.bfloat16
    q = jnp.concatenate([qa[0], qb[0]], axis=0)
    k = jnp.concatenate([ka[0], kb[0]], axis=0)
    v = jnp.concatenate([ua[0], ub[0]], axis=0)
    ohs = []
    for hh in range(_HG):
        sl = slice(hh * DH, (hh + 1) * DH)
        qh, kh, vh = q[:, sl], k[:, sl], v[:, sl]
        s = lax.dot_general(qh, kh, (((1,), (1,)), ((), ())),
                            preferred_element_type=f32)
        e = jnp.exp(s).astype(bf16)
        v_aug = jnp.concatenate([vh, jnp.ones_like(vh)], axis=1)
        oh_aug = jnp.dot(e, v_aug, preferred_element_type=f32)
        ohs.append((oh_aug[:, :DH] / oh_aug[:, DH:]).astype(bf16))
    acc = jnp.dot(jnp.concatenate(ohs, axis=1), wo[...],
                  preferred_element_type=f32)

    @pl.when(hg == 0)
    def _():
        out[0] = acc

    @pl.when(hg != 0)
    def _():
        out[0] = out[0] + acc


def _combine_kernel(v_in, u, out):
    j = pl.program_id(1)
    r = lax.broadcasted_iota(jnp.int32, (_RB, 1), 0) + j * _RB
    i = r // G
    sel = (r % G) == _seg(i)
    nsl = _RB // G
    u_exp = jnp.broadcast_to(u[0][:, None, :],
                             (nsl, G, VD)).reshape(_RB, VD)
    out[0] = v_in[0] + jnp.where(sel, u_exp, 0.0)


def _combine(values, u, half):
    jblocks = N // _RB
    nsl = _RB // G
    return pl.pallas_call(
        _combine_kernel,
        grid=(B, jblocks),
        in_specs=[
            pl.BlockSpec((1, _RB, VD), lambda b, j: (b, j, 0)),
            pl.BlockSpec((1, nsl, VD),
                         lambda b, j, h=half: (b, h * (K // nsl) + j, 0)),
        ],
        out_specs=pl.BlockSpec((1, _RB, VD), lambda b, j: (b, j, 0)),
        out_shape=jax.ShapeDtypeStruct((B, N, VD), jnp.float32),
        compiler_params=pltpu.CompilerParams(
            dimension_semantics=("arbitrary", "arbitrary"),
        ),
    )(values, u)


def kernel(values_a, metadata_a, values_b, metadata_b, Wq, Wk, Wv, Wo):
    bf16 = jnp.bfloat16

    full = lambda a: pl.BlockSpec(a.shape, lambda: (0,) * a.ndim)
    Wq16, Wk16, Wv16, Wo16 = pl.pallas_call(
        _cast_kernel,
        in_specs=[full(Wq), full(Wk), full(Wv), full(Wo)],
        out_specs=[full(Wq), full(Wk), full(Wv), full(Wo)],
        out_shape=[jax.ShapeDtypeStruct(a.shape, bf16)
                   for a in (Wq, Wk, Wv, Wo)],
    )(Wq, Wk, Wv, Wo)

    idx_flat = jnp.asarray(_IDX_FLAT)
    xva, xma = _sc_gather(values_a, metadata_a, idx_flat)
    xvb, xmb = _sc_gather(values_b, metadata_b, idx_flat)

    vspec = pl.BlockSpec((1, _CH, VD), lambda b, j: (b, j, 0))
    mspec = pl.BlockSpec((1, _CH, MD), lambda b, j: (b, j, 0))
    wspec = lambda r, c: pl.BlockSpec((r, c), lambda b, j: (0, 0))
    kid = jax.ShapeDtypeStruct((B, K, ID), bf16)
    qkv_out = pl.BlockSpec((1, _CH, ID), lambda b, j: (b, j, 0))

    def _project(xv, xm):
        return pl.pallas_call(
            _gp_kernel,
            grid=(B, K // _CH),
            in_specs=[
                vspec, mspec,
                wspec(VD + MD, ID), wspec(VD + MD, ID), wspec(VD, ID),
            ],
            out_specs=[qkv_out] * 3,
            out_shape=[kid] * 3,
            compiler_params=pltpu.CompilerParams(
                dimension_semantics=("arbitrary", "arbitrary"),
                vmem_limit_bytes=63 * 1024 * 1024,
            ),
        )(xv.reshape(B, K, VD), xm.reshape(B, K, MD), Wq16, Wk16, Wv16)

    qa, ka, ua = _project(xva, xma)
    qb, kb, ub = _project(xvb, xmb)

    hw = _HG * DH
    head_in = pl.BlockSpec((1, K, hw), lambda b, hg: (b, 0, hg))
    u = pl.pallas_call(
        _attn_kernel,
        grid=(B, H // _HG),
        in_specs=[head_in] * 6 + [
            pl.BlockSpec((hw, VD), lambda b, hg: (hg, 0)),
        ],
        out_specs=pl.BlockSpec((1, 2 * K, VD), lambda b, hg: (b, 0, 0)),
        out_shape=jax.ShapeDtypeStruct((B, 2 * K, VD), jnp.float32),
        compiler_params=pltpu.CompilerParams(
            dimension_semantics=("arbitrary", "arbitrary"),
            vmem_limit_bytes=63 * 1024 * 1024,
        ),
    )(qa, ka, ua, qb, kb, ub, Wo16)

    out_a = _combine(values_a, u, 0)
    out_b = _combine(values_b, u, 1)
    return out_a, out_b


# speedup vs baseline: 1.2244x; 1.0108x over previous
import functools

import numpy as np
import jax
import jax.numpy as jnp
from jax import lax
from jax.experimental import pallas as pl
from jax.experimental.pallas import tpu as pltpu
from jax.experimental.pallas import tpu_sc as plsc

B, N, VD, MD, ID, K, H = 4, 4096, 1024, 256, 1024, 1024, 16
DH = ID // H
G = N // K

_IDX = np.linspace(0, N - 1, K).astype(np.int64)
_OFF = _IDX - G * np.arange(K)
if not ((_OFF >= 0).all() and (_OFF < G).all()
        and np.all(np.isin(np.diff(_OFF), [0, 1]))):
    raise ValueError("anchor index structure unexpected")
_BOUNDS = [int(x) for x in (np.where(np.diff(_OFF) != 0)[0] + 1)]

_CH = 512
_HG = 4
_RB = 2048


def _seg(i):
    s = i * 0 if hasattr(i, "shape") else 0
    for bnd in _BOUNDS:
        s = s + (i >= bnd)
    return s


def _select_anchors(blk, j):
    d = blk.shape[-1]
    x4 = blk.reshape(_CH, G, d)
    i = lax.broadcasted_iota(jnp.int32, (_CH, 1), 0) + j * _CH
    seg = _seg(i)
    out = x4[:, 0, :] * (seg == 0).astype(blk.dtype)
    for o in range(1, G):
        out = out + x4[:, o, :] * (seg == o).astype(blk.dtype)
    return out


_IDX_FLAT = (np.arange(B)[:, None] * N + _IDX[None, :]).reshape(-1)
_IDX_FLAT = _IDX_FLAT.astype(np.int32)


def _sc_gather(values, metadata, idx_flat):
    info = plsc.get_sparse_core_info()
    nw = info.num_cores * info.num_subcores
    rows_total = B * K
    bpw = rows_total // nw
    cv = 64
    mesh = plsc.VectorSubcoreMesh(core_axis_name="c", subcore_axis_name="s")
    f32 = jnp.float32

    @functools.partial(
        pl.kernel, mesh=mesh,
        out_type=[jax.ShapeDtypeStruct((rows_total, VD), f32),
                  jax.ShapeDtypeStruct((rows_total, MD), f32)],
        scratch_types=[
            pltpu.VMEM((bpw,), jnp.int32),
            pltpu.VMEM((cv,), jnp.int32),
            pltpu.VMEM((cv, VD), f32),
            pltpu.VMEM((bpw, MD), f32),
            pltpu.SemaphoreType.DMA,
        ],
    )
    def gather(vt, mt, idx_hbm, vo, mo, idx_v, idx_c, rows_v, rows_m, sem):
        wid = lax.axis_index("s") * info.num_cores + lax.axis_index("c")
        base = wid * bpw
        pltpu.sync_copy(idx_hbm.at[pl.ds(base, bpw)], idx_v)
        for c in range(bpw // cv):
            pltpu.sync_copy(idx_hbm.at[pl.ds(base + c * cv, cv)], idx_c)
            pltpu.async_copy(vt.at[idx_c], rows_v, sem).wait()
            pltpu.sync_copy(rows_v, vo.at[pl.ds(base + c * cv, cv)])
        pltpu.async_copy(mt.at[idx_v], rows_m, sem).wait()
        pltpu.sync_copy(rows_m, mo.at[pl.ds(base, bpw)])

    return gather(values.reshape(B * N, VD),
                  metadata.reshape(B * N, MD),
                  idx_flat)


def _cast_kernel(wq, wk, wv, wo, wq16, wk16, wv16, wo16):
    bf16 = jnp.bfloat16
    wq16[...] = wq[...].astype(bf16)
    wk16[...] = wk[...].astype(bf16)
    wv16[...] = wv[...].astype(bf16)
    wo16[...] = wo[...].astype(bf16)


def _gp_kernel(v_in, m_in, wq, wk, wv, q_o, k_o, v_o):
    f32 = jnp.float32
    bf16 = jnp.bfloat16
    scale = 1.0 / np.sqrt(DH)
    wqv, wqm = wq[0:VD, :], wq[VD:, :]
    wkv, wkm = wk[0:VD, :], wk[VD:, :]
    xv = v_in[0].astype(bf16)
    xm = m_in[0].astype(bf16)
    q = (jnp.dot(xv, wqv, preferred_element_type=f32)
         + jnp.dot(xm, wqm, preferred_element_type=f32)) * scale
    k = (jnp.dot(xv, wkv, preferred_element_type=f32)
         + jnp.dot(xm, wkm, preferred_element_type=f32))
    v = jnp.dot(xv, wv[...], preferred_element_type=f32)
    q_o[0] = q.astype(bf16)
    k_o[0] = k.astype(bf16)
    v_o[0] = v.astype(bf16)


def _attn_kernel(qa, ka, ua, qb, kb, ub, wo, out):
    hg = pl.program_id(1)
    f32 = jnp.float32
    bf16 = jnp.bfloat16
    q = jnp.concatenate([qa[0], qb[0]], axis=0)
    k = jnp.concatenate([ka[0], kb[0]], axis=0)
    v = jnp.concatenate([ua[0], ub[0]], axis=0)
    ohs = []
    for hh in range(_HG):
        sl = slice(hh * DH, (hh + 1) * DH)
        qh, kh, vh = q[:, sl], k[:, sl], v[:, sl]
        s = lax.dot_general(qh, kh, (((1,), (1,)), ((), ())),
                            preferred_element_type=f32)
        e = jnp.exp(s).astype(bf16)
        v_aug = jnp.concatenate([vh, jnp.ones_like(vh)], axis=1)
        oh_aug = jnp.dot(e, v_aug, preferred_element_type=f32)
        ohs.append((oh_aug[:, :DH] / oh_aug[:, DH:]).astype(bf16))
    acc = jnp.dot(jnp.concatenate(ohs, axis=1), wo[...],
                  preferred_element_type=f32)

    @pl.when(hg == 0)
    def _():
        out[0] = acc

    @pl.when(hg != 0)
    def _():
        out[0] = out[0] + acc


def _combine_kernel(v_in, u, out):
    j = pl.program_id(1)
    r = lax.broadcasted_iota(jnp.int32, (_RB, 1), 0) + j * _RB
    i = r // G
    sel = (r % G) == _seg(i)
    nsl = _RB // G
    u_exp = jnp.broadcast_to(u[0][:, None, :],
                             (nsl, G, VD)).reshape(_RB, VD)
    out[0] = v_in[0] + jnp.where(sel, u_exp, 0.0)


def _combine(values, u, half):
    jblocks = N // _RB
    nsl = _RB // G
    return pl.pallas_call(
        _combine_kernel,
        grid=(B, jblocks),
        in_specs=[
            pl.BlockSpec((1, _RB, VD), lambda b, j: (b, j, 0)),
            pl.BlockSpec((1, nsl, VD),
                         lambda b, j, h=half: (b, h * (K // nsl) + j, 0)),
        ],
        out_specs=pl.BlockSpec((1, _RB, VD), lambda b, j: (b, j, 0)),
        out_shape=jax.ShapeDtypeStruct((B, N, VD), jnp.float32),
        compiler_params=pltpu.CompilerParams(
            dimension_semantics=("arbitrary", "arbitrary"),
        ),
    )(values, u)


def kernel(values_a, metadata_a, values_b, metadata_b, Wq, Wk, Wv, Wo):
    bf16 = jnp.bfloat16

    full = lambda a: pl.BlockSpec(a.shape, lambda: (0,) * a.ndim)
    Wq16, Wk16, Wv16, Wo16 = pl.pallas_call(
        _cast_kernel,
        in_specs=[full(Wq), full(Wk), full(Wv), full(Wo)],
        out_specs=[full(Wq), full(Wk), full(Wv), full(Wo)],
        out_shape=[jax.ShapeDtypeStruct(a.shape, bf16)
                   for a in (Wq, Wk, Wv, Wo)],
    )(Wq, Wk, Wv, Wo)

    idx_flat = jnp.asarray(_IDX_FLAT)
    xva, xma = _sc_gather(values_a, metadata_a, idx_flat)
    xvb, xmb = _sc_gather(values_b, metadata_b, idx_flat)

    vspec = pl.BlockSpec((1, _CH, VD), lambda b, j: (b, j, 0))
    mspec = pl.BlockSpec((1, _CH, MD), lambda b, j: (b, j, 0))
    wspec = lambda r, c: pl.BlockSpec((r, c), lambda b, j: (0, 0))
    kid = jax.ShapeDtypeStruct((B, K, ID), bf16)
    qkv_out = pl.BlockSpec((1, _CH, ID), lambda b, j: (b, j, 0))

    def _project(xv, xm):
        return pl.pallas_call(
            _gp_kernel,
            grid=(B, K // _CH),
            in_specs=[
                vspec, mspec,
                wspec(VD + MD, ID), wspec(VD + MD, ID), wspec(VD, ID),
            ],
            out_specs=[qkv_out] * 3,
            out_shape=[kid] * 3,
            compiler_params=pltpu.CompilerParams(
                dimension_semantics=("arbitrary", "arbitrary"),
                vmem_limit_bytes=63 * 1024 * 1024,
            ),
        )(xv.reshape(B, K, VD), xm.reshape(B, K, MD), Wq16, Wk16, Wv16)

    qa, ka, ua = _project(xva, xma)
    qb, kb, ub = _project(xvb, xmb)

    hw = _HG * DH
    head_in = pl.BlockSpec((1, K, hw), lambda b, hg: (b, 0, hg))
    u = pl.pallas_call(
        _attn_kernel,
        grid=(B, H // _HG),
        in_specs=[head_in] * 6 + [
            pl.BlockSpec((hw, VD), lambda b, hg: (hg, 0)),
        ],
        out_specs=pl.BlockSpec((1, 2 * K, VD), lambda b, hg: (b, 0, 0)),
        out_shape=jax.ShapeDtypeStruct((B, 2 * K, VD), jnp.float32),
        compiler_params=pltpu.CompilerParams(
            dimension_semantics=("arbitrary", "arbitrary"),
            vmem_limit_bytes=63 * 1024 * 1024,
        ),
    )(qa, ka, ua, qb, kb, ub, Wo16)

    out_a = _combine(values_a, u, 0)
    out_b = _combine(values_b, u, 1)
    return out_a, out_b


# speedup vs baseline: 1.2284x; 1.0033x over previous
import functools

import numpy as np
import jax
import jax.numpy as jnp
from jax import lax
from jax.experimental import pallas as pl
from jax.experimental.pallas import tpu as pltpu
from jax.experimental.pallas import tpu_sc as plsc

B, N, VD, MD, ID, K, H = 4, 4096, 1024, 256, 1024, 1024, 16
DH = ID // H
G = N // K

_IDX = np.linspace(0, N - 1, K).astype(np.int64)
_OFF = _IDX - G * np.arange(K)
if not ((_OFF >= 0).all() and (_OFF < G).all()
        and np.all(np.isin(np.diff(_OFF), [0, 1]))):
    raise ValueError("anchor index structure unexpected")
_BOUNDS = [int(x) for x in (np.where(np.diff(_OFF) != 0)[0] + 1)]

_CH = 512
_HG = 4
_RB = 2048


def _seg(i):
    s = i * 0 if hasattr(i, "shape") else 0
    for bnd in _BOUNDS:
        s = s + (i >= bnd)
    return s


_IDX_FLAT = (np.arange(B)[:, None] * N + _IDX[None, :]).reshape(-1)
_IDX_FLAT = _IDX_FLAT.astype(np.int32)


def _sc_gather(values, metadata, idx_flat):
    info = plsc.get_sparse_core_info()
    nw = info.num_cores * info.num_subcores
    rows_total = B * K
    bpw = rows_total // nw
    cv = 64
    mesh = plsc.VectorSubcoreMesh(core_axis_name="c", subcore_axis_name="s")
    f32 = jnp.float32

    @functools.partial(
        pl.kernel, mesh=mesh,
        out_type=[jax.ShapeDtypeStruct((rows_total, VD), f32),
                  jax.ShapeDtypeStruct((rows_total, MD), f32)],
        scratch_types=[
            pltpu.VMEM((bpw,), jnp.int32),
            pltpu.VMEM((cv,), jnp.int32),
            pltpu.VMEM((cv, VD), f32),
            pltpu.VMEM((bpw, MD), f32),
            pltpu.SemaphoreType.DMA,
        ],
    )
    def gather(vt, mt, idx_hbm, vo, mo, idx_v, idx_c, rows_v, rows_m, sem):
        wid = lax.axis_index("s") * info.num_cores + lax.axis_index("c")
        base = wid * bpw
        pltpu.sync_copy(idx_hbm.at[pl.ds(base, bpw)], idx_v)
        for c in range(bpw // cv):
            pltpu.sync_copy(idx_hbm.at[pl.ds(base + c * cv, cv)], idx_c)
            pltpu.async_copy(vt.at[idx_c], rows_v, sem).wait()
            pltpu.sync_copy(rows_v, vo.at[pl.ds(base + c * cv, cv)])
        pltpu.async_copy(mt.at[idx_v], rows_m, sem).wait()
        pltpu.sync_copy(rows_m, mo.at[pl.ds(base, bpw)])

    return gather(values.reshape(B * N, VD),
                  metadata.reshape(B * N, MD),
                  idx_flat)


def _cast_kernel(wq, wk, wv, wo, wq16, wk16, wv16, wo16):
    bf16 = jnp.bfloat16
    wq16[...] = wq[...].astype(bf16)
    wk16[...] = wk[...].astype(bf16)
    wv16[...] = wv[...].astype(bf16)
    wo16[...] = wo[...].astype(bf16)


def _gp_kernel(v_in, m_in, wq, wk, wv, q_o, k_o, v_o):
    f32 = jnp.float32
    bf16 = jnp.bfloat16
    scale = 1.0 / np.sqrt(DH)
    wqv, wqm = wq[0:VD, :], wq[VD:, :]
    wkv, wkm = wk[0:VD, :], wk[VD:, :]
    xv = v_in[0].astype(bf16)
    xm = m_in[0].astype(bf16)
    q = (jnp.dot(xv, wqv, preferred_element_type=f32)
         + jnp.dot(xm, wqm, preferred_element_type=f32)) * scale
    k = (jnp.dot(xv, wkv, preferred_element_type=f32)
         + jnp.dot(xm, wkm, preferred_element_type=f32))
    v = jnp.dot(xv, wv[...], preferred_element_type=f32)
    q_o[0] = q.astype(bf16)
    k_o[0] = k.astype(bf16)
    v_o[0] = v.astype(bf16)


def _attn_kernel(qa, ka, ua, qb, kb, ub, wo, out):
    hg = pl.program_id(1)
    f32 = jnp.float32
    bf16 = jnp.bfloat16
    q = jnp.concatenate([qa[0], qb[0]], axis=0)
    k = jnp.concatenate([ka[0], kb[0]], axis=0)
    v = jnp.concatenate([ua[0], ub[0]], axis=0)
    ohs = []
    for hh in range(_HG):
        sl = slice(hh * DH, (hh + 1) * DH)
        qh, kh, vh = q[:, sl], k[:, sl], v[:, sl]
        s = lax.dot_general(qh, kh, (((1,), (1,)), ((), ())),
                            preferred_element_type=f32)
        e = jnp.exp(s).astype(bf16)
        v_aug = jnp.concatenate([vh, jnp.ones_like(vh)], axis=1)
        oh_aug = jnp.dot(e, v_aug, preferred_element_type=f32)
        ohs.append((oh_aug[:, :DH] / oh_aug[:, DH:]).astype(bf16))
    acc = jnp.dot(jnp.concatenate(ohs, axis=1), wo[...],
                  preferred_element_type=f32)

    @pl.when(hg == 0)
    def _():
        out[0] = acc

    @pl.when(hg != 0)
    def _():
        out[0] = out[0] + acc


def _combine_kernel(v_in, u, out):
    j = pl.program_id(1)
    r = lax.broadcasted_iota(jnp.int32, (_RB, 1), 0) + j * _RB
    i = r // G
    sel = (r % G) == _seg(i)
    nsl = _RB // G
    u_exp = jnp.broadcast_to(u[0][:, None, :],
                             (nsl, G, VD)).reshape(_RB, VD)
    out[0] = v_in[0] + jnp.where(sel, u_exp, 0.0)


def _combine(values, u, half):
    jblocks = N // _RB
    nsl = _RB // G
    return pl.pallas_call(
        _combine_kernel,
        grid=(B, jblocks),
        in_specs=[
            pl.BlockSpec((1, _RB, VD), lambda b, j: (b, j, 0)),
            pl.BlockSpec((1, nsl, VD),
                         lambda b, j, h=half: (b, h * (K // nsl) + j, 0)),
        ],
        out_specs=pl.BlockSpec((1, _RB, VD), lambda b, j: (b, j, 0)),
        out_shape=jax.ShapeDtypeStruct((B, N, VD), jnp.float32),
        compiler_params=pltpu.CompilerParams(
            dimension_semantics=("arbitrary", "arbitrary"),
        ),
    )(values, u)


def kernel(values_a, metadata_a, values_b, metadata_b, Wq, Wk, Wv, Wo):
    bf16 = jnp.bfloat16

    full = lambda a: pl.BlockSpec(a.shape, lambda: (0,) * a.ndim)
    Wq16, Wk16, Wv16, Wo16 = pl.pallas_call(
        _cast_kernel,
        in_specs=[full(Wq), full(Wk), full(Wv), full(Wo)],
        out_specs=[full(Wq), full(Wk), full(Wv), full(Wo)],
        out_shape=[jax.ShapeDtypeStruct(a.shape, bf16)
                   for a in (Wq, Wk, Wv, Wo)],
    )(Wq, Wk, Wv, Wo)

    idx_flat = jnp.asarray(_IDX_FLAT)
    xva, xma = _sc_gather(values_a, metadata_a, idx_flat)
    xvb, xmb = _sc_gather(values_b, metadata_b, idx_flat)

    vspec = pl.BlockSpec((1, _CH, VD), lambda b, j: (b, j, 0))
    mspec = pl.BlockSpec((1, _CH, MD), lambda b, j: (b, j, 0))
    wspec = lambda r, c: pl.BlockSpec((r, c), lambda b, j: (0, 0))
    kid = jax.ShapeDtypeStruct((B, K, ID), bf16)
    qkv_out = pl.BlockSpec((1, _CH, ID), lambda b, j: (b, j, 0))

    def _project(xv, xm):
        return pl.pallas_call(
            _gp_kernel,
            grid=(B, K // _CH),
            in_specs=[
                vspec, mspec,
                wspec(VD + MD, ID), wspec(VD + MD, ID), wspec(VD, ID),
            ],
            out_specs=[qkv_out] * 3,
            out_shape=[kid] * 3,
            compiler_params=pltpu.CompilerParams(
                dimension_semantics=("arbitrary", "arbitrary"),
                vmem_limit_bytes=63 * 1024 * 1024,
            ),
        )(xv.reshape(B, K, VD), xm.reshape(B, K, MD), Wq16, Wk16, Wv16)

    qa, ka, ua = _project(xva, xma)
    qb, kb, ub = _project(xvb, xmb)

    hw = _HG * DH
    head_in = pl.BlockSpec((1, K, hw), lambda b, hg: (b, 0, hg))
    u = pl.pallas_call(
        _attn_kernel,
        grid=(B, H // _HG),
        in_specs=[head_in] * 6 + [
            pl.BlockSpec((hw, VD), lambda b, hg: (hg, 0)),
        ],
        out_specs=pl.BlockSpec((1, 2 * K, VD), lambda b, hg: (b, 0, 0)),
        out_shape=jax.ShapeDtypeStruct((B, 2 * K, VD), jnp.float32),
        compiler_params=pltpu.CompilerParams(
            dimension_semantics=("arbitrary", "arbitrary"),
            vmem_limit_bytes=63 * 1024 * 1024,
        ),
    )(qa, ka, ua, qb, kb, ub, Wo16)

    out_a = _combine(values_a, u, 0)
    out_b = _combine(values_b, u, 1)
    return out_a, out_b
